# baseline (speedup 1.0000x reference)
import jax
import jax.numpy as jnp
from jax import lax
from jax.experimental import pallas as pl
from jax.experimental.pallas import tpu as pltpu
from jax.experimental.pallas import tpu_sc as plsc

N_TOK = 16384
N_CODE = 1024
DIM = 256
BZ = 512

N_PHASE = 2
TOK_PH = N_TOK // N_PHASE
NB = TOK_PH // BZ

_SC_INFO = plsc.get_sparse_core_info()
NC = _SC_INFO.num_cores
NS = _SC_INFO.num_subcores
NW = NC * NS
B_PER_W = TOK_PH // NW
CH = 128
N_CHUNK = B_PER_W // CH


def _vq_argmin_body(z_ref, w_ref, idx_ref, loss_ref):
    z = z_ref[...]
    w = w_ref[...]
    zsq = jnp.sum(z * z, axis=1, keepdims=True)
    wsq = jnp.sum(w * w, axis=1)
    cross = lax.dot_general(z, w, (((1,), (1,)), ((), ())),
                            preferred_element_type=jnp.float32)
    dist = zsq + wsq[None, :] - 2.0 * cross
    minval = jnp.min(dist, axis=1, keepdims=True)
    col = lax.broadcasted_iota(jnp.int32, (BZ, N_CODE), 1)
    idx = jnp.min(jnp.where(dist == minval, col, N_CODE), axis=1)
    idx_ref[0, 0, :] = idx

    part = jnp.sum(minval).reshape(1, 1)

    @pl.when(pl.program_id(0) == 0)
    def _():
        loss_ref[...] = jnp.zeros((1, 1), jnp.float32)

    loss_ref[...] += part


def _sc_gather_body(w_hbm, idx_hbm, out_hbm, idx_v, rows_v, gsem, ssem):
    wid = lax.axis_index("s") * NC + lax.axis_index("c")
    base = wid * B_PER_W
    pltpu.sync_copy(idx_hbm.at[pl.ds(base, B_PER_W)], idx_v)
    gathers = [None, None]
    stores = [None, None]
    gathers[0] = pltpu.async_copy(
        w_hbm.at[idx_v.at[pl.ds(0, CH)]], rows_v.at[0], gsem)
    for c in range(N_CHUNK):
        b = c % 2
        nb = (c + 1) % 2
        if c + 1 < N_CHUNK:
            if stores[nb] is not None:
                stores[nb].wait()
            gathers[nb] = pltpu.async_copy(
                w_hbm.at[idx_v.at[pl.ds((c + 1) * CH, CH)]],
                rows_v.at[nb], gsem)
        gathers[b].wait()
        stores[b] = pltpu.async_copy(
            rows_v.at[b], out_hbm.at[pl.ds(base + c * CH, CH)], ssem)
    for st in stores:
        if st is not None:
            st.wait()


def _tc_argmin(z_half, W):
    return pl.pallas_call(
        _vq_argmin_body,
        grid=(NB,),
        in_specs=[
            pl.BlockSpec((BZ, DIM), lambda i: (i, 0)),
            pl.BlockSpec((N_CODE, DIM), lambda i: (0, 0)),
        ],
        out_specs=[
            pl.BlockSpec((1, 1, BZ), lambda i: (i, 0, 0)),
            pl.BlockSpec((1, 1), lambda i: (0, 0)),
        ],
        out_shape=[
            jax.ShapeDtypeStruct((NB, 1, BZ), jnp.int32),
            jax.ShapeDtypeStruct((1, 1), jnp.float32),
        ],
    )(z_half, W)


_sc_gather = None


def _make_sc_gather():
    return pl.kernel(
        _sc_gather_body,
        mesh=plsc.VectorSubcoreMesh(core_axis_name="c", subcore_axis_name="s"),
        out_type=jax.ShapeDtypeStruct((TOK_PH, DIM), jnp.float32),
        scratch_types=[
            pltpu.VMEM((B_PER_W,), jnp.int32),
            pltpu.VMEM((2, CH, DIM), jnp.float32),
            pltpu.SemaphoreType.DMA,
            pltpu.SemaphoreType.DMA,
        ],
    )


def kernel(z_e, W):
    gather = _make_sc_gather()
    zq_parts = []
    loss_total = None
    indices_parts = []
    for p in range(N_PHASE):
        z_half = lax.slice_in_dim(z_e, p * TOK_PH, (p + 1) * TOK_PH, axis=0)
        idx3, loss = _tc_argmin(z_half, W)
        idx_flat = idx3.reshape(TOK_PH)
        indices_parts.append(idx_flat)
        zq_parts.append(gather(W, idx_flat))
        loss_total = loss if loss_total is None else loss_total + loss

    z_q = jnp.concatenate(zq_parts, axis=0)
    indices = jnp.concatenate(indices_parts, axis=0)
    vq_loss = loss_total[0, 0] * (1.25 / (N_TOK * DIM))
    return (z_q, indices, vq_loss)


# speedup vs baseline: 1.2386x; 1.2386x over previous
import jax
import jax.numpy as jnp
from jax import lax
from jax.experimental import pallas as pl
from jax.experimental.pallas import tpu as pltpu
from jax.experimental.pallas import tpu_sc as plsc

N_TOK = 16384
N_CODE = 1024
DIM = 256
BZ = 512
NB = N_TOK // BZ

_SC_INFO = plsc.get_sparse_core_info()
NC = _SC_INFO.num_cores
NS = _SC_INFO.num_subcores
NW = NC * NS
B_PER_W = N_TOK // NW
CH = 128
N_CHUNK = B_PER_W // CH


def _vq_argmin_body(z_ref, w_ref, idx_ref, loss_ref):
    z = z_ref[...]
    w = w_ref[...]
    zsq = jnp.sum(z * z, axis=1, keepdims=True)
    wsq = jnp.sum(w * w, axis=1)
    cross = lax.dot_general(z, w, (((1,), (1,)), ((), ())),
                            preferred_element_type=jnp.float32)
    dist = zsq + wsq[None, :] - 2.0 * cross
    minval = jnp.min(dist, axis=1, keepdims=True)
    col = lax.broadcasted_iota(jnp.int32, (BZ, N_CODE), 1)
    idx = jnp.min(jnp.where(dist == minval, col, N_CODE), axis=1)
    idx_ref[0, 0, :] = idx

    part = jnp.sum(minval).reshape(1, 1)

    @pl.when(pl.program_id(0) == 0)
    def _():
        loss_ref[...] = jnp.zeros((1, 1), jnp.float32)

    loss_ref[...] += part


def _sc_gather_body(w_hbm, idx_hbm, out_hbm, idx_v, rows_v, gsem, ssem):
    wid = lax.axis_index("s") * NC + lax.axis_index("c")
    base = wid * B_PER_W
    pltpu.sync_copy(idx_hbm.at[pl.ds(base, B_PER_W)], idx_v)
    gathers = [None, None]
    stores = [None, None]
    gathers[0] = pltpu.async_copy(
        w_hbm.at[idx_v.at[pl.ds(0, CH)]], rows_v.at[0], gsem)
    for c in range(N_CHUNK):
        b = c % 2
        nb = (c + 1) % 2
        if c + 1 < N_CHUNK:
            if stores[nb] is not None:
                stores[nb].wait()
            gathers[nb] = pltpu.async_copy(
                w_hbm.at[idx_v.at[pl.ds((c + 1) * CH, CH)]],
                rows_v.at[nb], gsem)
        gathers[b].wait()
        stores[b] = pltpu.async_copy(
            rows_v.at[b], out_hbm.at[pl.ds(base + c * CH, CH)], ssem)
    for st in stores:
        if st is not None:
            st.wait()


def _tc_argmin(z_e, W):
    return pl.pallas_call(
        _vq_argmin_body,
        grid=(NB,),
        in_specs=[
            pl.BlockSpec((BZ, DIM), lambda i: (i, 0)),
            pl.BlockSpec((N_CODE, DIM), lambda i: (0, 0)),
        ],
        out_specs=[
            pl.BlockSpec((1, 1, BZ), lambda i: (i, 0, 0)),
            pl.BlockSpec((1, 1), lambda i: (0, 0)),
        ],
        out_shape=[
            jax.ShapeDtypeStruct((NB, 1, BZ), jnp.int32),
            jax.ShapeDtypeStruct((1, 1), jnp.float32),
        ],
    )(z_e, W)


def _make_sc_gather():
    return pl.kernel(
        _sc_gather_body,
        mesh=plsc.VectorSubcoreMesh(core_axis_name="c", subcore_axis_name="s"),
        out_type=jax.ShapeDtypeStruct((N_TOK, DIM), jnp.float32),
        scratch_types=[
            pltpu.VMEM((B_PER_W,), jnp.int32),
            pltpu.VMEM((2, CH, DIM), jnp.float32),
            pltpu.SemaphoreType.DMA,
            pltpu.SemaphoreType.DMA,
        ],
    )


def kernel(z_e, W):
    idx3, loss = _tc_argmin(z_e, W)
    indices = idx3.reshape(N_TOK)
    z_q = _make_sc_gather()(W, indices)
    vq_loss = loss[0, 0] * (1.25 / (N_TOK * DIM))
    return (z_q, indices, vq_loss)
